# baseline (speedup 1.0000x reference)
import functools

import jax
import jax.numpy as jnp
from jax import lax
from jax.experimental import pallas as pl
from jax.experimental.pallas import tpu as pltpu
from jax.experimental.pallas import tpu_sc as plsc

N_NODES = 50000
D = 128
DW = D // 2
S = 10
BATCH = 50000

NC = 2
NS = 16
NW = NC * NS

CH = 128
CPW = 13
PER_W = CH * CPW
PAD = NW * PER_W

G = 64
NCHUNK = PER_W // G

TB = 2048


def _worker_id():
    return lax.axis_index("s") * NC + lax.axis_index("c")


def _s1_issue(feat_hbm, idx_v, buf, sem, g):
    gg, half = g // 2, g % 2
    drains = []
    for s in range(S):
        idxs = idx_v.at[s * CPW + gg, pl.ds(half * G, G)]
        drains.append(pltpu.async_copy(
            feat_hbm.at[idxs], buf.at[pl.ds(s * G, G)], sem))
    return drains


def _s1_reduce(gb, ob):
    himask = jnp.int32(-65536)
    half = jnp.int32(0x8000)

    def body(r, _):
        for c in range(DW // 16):
            col = pl.ds(c * 16, 16)
            x = gb[r, col]
            lo = plsc.bitcast(x << 16, jnp.float32)
            hi = plsc.bitcast(x & himask, jnp.float32)
            for s in range(1, S):
                x = gb[s * G + r, col]
                lo = lo + plsc.bitcast(x << 16, jnp.float32)
                hi = hi + plsc.bitcast(x & himask, jnp.float32)
            loi = plsc.bitcast(lo, jnp.int32)
            hii = plsc.bitcast(hi, jnp.int32)
            ob[r, col] = (lax.shift_right_logical(loi + half, 16)
                          | ((hii + half) & himask))
        return 0

    lax.fori_loop(0, G, body, 0)


def _s1_body(nidx_hbm, feat_hbm, nsum_hbm, idx_v, gb0, gb1, ob0, ob1,
             gsem0, gsem1, osem0, osem1):
    wid = _worker_id()
    base_r = wid * PER_W

    pltpu.sync_copy(nidx_hbm.at[wid], idx_v)

    gbufs = (gb0, gb1)
    obufs = (ob0, ob1)
    gsems = (gsem0, gsem1)
    osems = (osem0, osem1)
    ostore = [None, None]
    pending = _s1_issue(feat_hbm, idx_v, gb0, gsem0, 0)
    for g in range(NCHUNK):
        b = g % 2
        nxt = None
        if g + 1 < NCHUNK:
            nxt = _s1_issue(feat_hbm, idx_v, gbufs[1 - b], gsems[1 - b],
                            g + 1)
        for d in pending:
            d.wait()
        if ostore[b] is not None:
            ostore[b].wait()
        _s1_reduce(gbufs[b], obufs[b])
        ostore[b] = pltpu.async_copy(
            obufs[b], nsum_hbm.at[pl.ds(base_r + g * G, G)], osems[b])
        pending = nxt
    for d in ostore:
        if d is not None:
            d.wait()


@functools.partial(
    pl.kernel,
    out_type=jax.ShapeDtypeStruct((PAD, DW), jnp.int32),
    mesh=plsc.VectorSubcoreMesh(core_axis_name="c", subcore_axis_name="s"),
    compiler_params=pltpu.CompilerParams(use_tc_tiling_on_sc=False,
                                         needs_layout_passes=False),
    scratch_types=[
        pltpu.VMEM((S * CPW, CH), jnp.int32),
        pltpu.VMEM((S * G, DW), jnp.int32),
        pltpu.VMEM((S * G, DW), jnp.int32),
        pltpu.VMEM((G, DW), jnp.int32),
        pltpu.VMEM((G, DW), jnp.int32),
        pltpu.SemaphoreType.DMA,
        pltpu.SemaphoreType.DMA,
        pltpu.SemaphoreType.DMA,
        pltpu.SemaphoreType.DMA,
    ],
)
def _stage1(nidx_hbm, feat_hbm, nsum_hbm, idx_v, gb0, gb1, ob0, ob1,
            gsem0, gsem1, osem0, osem1):
    _s1_body(nidx_hbm, feat_hbm, nsum_hbm, idx_v, gb0, gb1, ob0, ob1,
             gsem0, gsem1, osem0, osem1)


def _s2_body(nodes_hbm, pw_hbm, out_hbm, nodes_v, buf, gsem):
    wid = _worker_id()
    base_r = wid * PER_W

    pltpu.sync_copy(nodes_hbm.at[wid], nodes_v)

    drains = []
    for g in range(CPW):
        drains.append(pltpu.async_copy(
            pw_hbm.at[nodes_v.at[g]], buf.at[pl.ds(g * CH, CH)], gsem))
    for d in drains:
        d.wait()
    pltpu.sync_copy(buf, out_hbm.at[pl.ds(base_r, PER_W)])


@functools.partial(
    pl.kernel,
    out_type=jax.ShapeDtypeStruct((PAD, DW), jnp.int32),
    mesh=plsc.VectorSubcoreMesh(core_axis_name="c", subcore_axis_name="s"),
    compiler_params=pltpu.CompilerParams(use_tc_tiling_on_sc=False),
    scratch_types=[
        pltpu.VMEM((CPW, CH), jnp.int32),
        pltpu.VMEM((PER_W, DW), jnp.int32),
        pltpu.SemaphoreType.DMA,
    ],
)
def _stage2(nodes_hbm, pw_hbm, out_hbm, nodes_v, buf, gsem):
    _s2_body(nodes_hbm, pw_hbm, out_hbm, nodes_v, buf, gsem)


def _prep_body(wi_ref, wf_ref, bi_ref, bf_ref, a_ref, c_ref, c0_ref):
    wi = wi_ref[...]
    wf1 = wf_ref[:D, :]
    wf2 = wf_ref[D:, :]
    a_ref[...] = jnp.dot(wi, wf1, preferred_element_type=jnp.float32)
    c_ref[...] = jnp.dot(wi, wf2, preferred_element_type=jnp.float32) * (
        1.0 / S)
    c0_ref[...] = (jnp.dot(bi_ref[...], wf1 + wf2,
                           preferred_element_type=jnp.float32) + bf_ref[...])


def _mm_body(feat_ref, nsum_ref, a_ref, c_ref, c0_ref, o_ref):
    nsum = nsum_ref[...].astype(jnp.float32)
    x = jnp.dot(feat_ref[...], a_ref[...], preferred_element_type=jnp.float32)
    x = x + jnp.dot(nsum, c_ref[...], preferred_element_type=jnp.float32)
    x = x + c0_ref[...]
    o_ref[...] = (x * (1.0 / (1.0 + jnp.exp(-x)))).astype(jnp.bfloat16)


def _fold_weights(w_init, w_final, b_init, b_final):
    return pl.pallas_call(
        _prep_body,
        out_shape=(jax.ShapeDtypeStruct((D, D), jnp.float32),
                   jax.ShapeDtypeStruct((D, D), jnp.float32),
                   jax.ShapeDtypeStruct((1, D), jnp.float32)),
    )(w_init, w_final, b_init.reshape(1, D), b_final.reshape(1, D))


def _matmul_swish(feats_p, nsum_bf, a, c, c0):
    grid = (PAD // TB,)
    return pl.pallas_call(
        _mm_body,
        grid=grid,
        in_specs=[
            pl.BlockSpec((TB, D), lambda i: (i, 0)),
            pl.BlockSpec((TB, D), lambda i: (i, 0)),
            pl.BlockSpec((D, D), lambda i: (0, 0)),
            pl.BlockSpec((D, D), lambda i: (0, 0)),
            pl.BlockSpec((1, D), lambda i: (0, 0)),
        ],
        out_specs=pl.BlockSpec((TB, D), lambda i: (i, 0)),
        out_shape=jax.ShapeDtypeStruct((PAD, D), jnp.bfloat16),
    )(feats_p, nsum_bf, a, c, c0)


@jax.jit
def kernel(nodes, neigh_idx, features, W_init, b_init, W_final, b_final):
    nodes_p = jnp.pad(nodes.astype(jnp.int32), (0, PAD - BATCH))
    nidx_t = jnp.pad(neigh_idx.astype(jnp.int32),
                     ((0, PAD - N_NODES), (0, 0))).T
    nidx_w = nidx_t.reshape(S, NW, CPW * CH).transpose(1, 0, 2).reshape(
        NW, S * CPW, CH)
    nodes3 = nodes_p.reshape(NW, CPW, CH)
    feats_p = jnp.pad(features, ((0, PAD - N_NODES), (0, 0)))
    feats_i = lax.bitcast_convert_type(
        features.astype(jnp.bfloat16).reshape(N_NODES, DW, 2), jnp.int32)

    nsum_i = _stage1(nidx_w, feats_i)
    nsum_bf = lax.bitcast_convert_type(nsum_i, jnp.bfloat16).reshape(PAD, D)
    a, c, c0 = _fold_weights(W_init, W_final, b_init, b_final)
    pw_bf = _matmul_swish(feats_p, nsum_bf, a, c, c0)
    pw_i = lax.bitcast_convert_type(pw_bf.reshape(PAD, DW, 2), jnp.int32)
    out_i = _stage2(nodes3, pw_i)
    out_bf = lax.bitcast_convert_type(out_i, jnp.bfloat16).reshape(PAD, D)
    return out_bf[:BATCH].astype(jnp.float32)


# speedup vs baseline: 1.0191x; 1.0191x over previous
import functools

import jax
import jax.numpy as jnp
from jax import lax
from jax.experimental import pallas as pl
from jax.experimental.pallas import tpu as pltpu
from jax.experimental.pallas import tpu_sc as plsc

N_NODES = 50000
D = 128
S = 10
BATCH = 50000

NC = 2
NS = 16
NW = NC * NS

CH = 128
CPW = 13
PER_W = CH * CPW
PAD = NW * PER_W

PLAN = ((0, 4, 0), (4, 2, 1), (6, 4, 0), (10, 2, 1), (12, 1, 1))

TB = 2048


def _worker_id():
    return lax.axis_index("s") * NC + lax.axis_index("c")


def _zero_rows(acc, nrows):
    zero = jnp.zeros((16,), jnp.float32)

    def body(r, _):
        for c in range(D // 16):
            acc[r, pl.ds(c * 16, 16)] = zero
        return 0

    lax.fori_loop(0, nrows, body, 0)


def _s1_body(nidx_hbm, feat_hbm, nsum_hbm, idx_v, acc0_v, acc1_v,
             gsem0, gsem1, ssem0, ssem1):
    wid = _worker_id()
    base_c = wid * CPW

    pltpu.sync_copy(nidx_hbm.at[wid], idx_v)

    accs = (acc0_v, acc1_v)
    gsems = (gsem0, gsem1)
    ssems = (ssem0, ssem1)
    store = [None, None]
    for g0, ck, b in PLAN:
        if store[b] is not None:
            store[b].wait()
        acc = accs[b]
        _zero_rows(acc, ck * CH)
        drains = []
        for c in range(ck):
            dst = acc.at[pl.ds(c * CH, CH)]
            for s in range(S):
                drains.append(pltpu.async_copy(
                    feat_hbm.at[idx_v.at[s * CPW + g0 + c]], dst, gsems[b],
                    add=True))
        for d in drains:
            d.wait()
        store[b] = pltpu.async_copy(
            acc.at[pl.ds(0, ck * CH)],
            nsum_hbm.at[pl.ds((base_c + g0) * CH, ck * CH)], ssems[b])
    for d in store:
        if d is not None:
            d.wait()


@functools.partial(
    pl.kernel,
    out_type=jax.ShapeDtypeStruct((PAD, D), jnp.float32),
    mesh=plsc.VectorSubcoreMesh(core_axis_name="c", subcore_axis_name="s"),
    scratch_types=[
        pltpu.VMEM((S * CPW, CH), jnp.int32),
        pltpu.VMEM((4 * CH, D), jnp.float32),
        pltpu.VMEM((2 * CH, D), jnp.float32),
        pltpu.SemaphoreType.DMA,
        pltpu.SemaphoreType.DMA,
        pltpu.SemaphoreType.DMA,
        pltpu.SemaphoreType.DMA,
    ],
)
def _stage1(nidx_hbm, feat_hbm, nsum_hbm, idx_v, acc0_v, acc1_v,
            gsem0, gsem1, ssem0, ssem1):
    _s1_body(nidx_hbm, feat_hbm, nsum_hbm, idx_v, acc0_v, acc1_v,
             gsem0, gsem1, ssem0, ssem1)


def _s2_body(nodes_hbm, pw_hbm, out_hbm, nodes_v, buf0, buf1,
             gsem0, gsem1, ssem0, ssem1):
    wid = _worker_id()
    base_c = wid * CPW

    pltpu.sync_copy(nodes_hbm.at[wid], nodes_v)

    bufs = (buf0, buf1)
    gsems = (gsem0, gsem1)
    ssems = (ssem0, ssem1)
    store = [None, None]
    for g0, ck, b in PLAN:
        if store[b] is not None:
            store[b].wait()
        buf = bufs[b]
        drains = []
        for c in range(ck):
            drains.append(pltpu.async_copy(
                pw_hbm.at[nodes_v.at[g0 + c]],
                buf.at[pl.ds(c * CH, CH)], gsems[b]))
        for d in drains:
            d.wait()
        store[b] = pltpu.async_copy(
            buf.at[pl.ds(0, ck * CH)],
            out_hbm.at[pl.ds((base_c + g0) * CH, ck * CH)], ssems[b])
    for d in store:
        if d is not None:
            d.wait()


@functools.partial(
    pl.kernel,
    out_type=jax.ShapeDtypeStruct((PAD, D), jnp.float32),
    mesh=plsc.VectorSubcoreMesh(core_axis_name="c", subcore_axis_name="s"),
    scratch_types=[
        pltpu.VMEM((CPW, CH), jnp.int32),
        pltpu.VMEM((4 * CH, D), jnp.float32),
        pltpu.VMEM((2 * CH, D), jnp.float32),
        pltpu.SemaphoreType.DMA,
        pltpu.SemaphoreType.DMA,
        pltpu.SemaphoreType.DMA,
        pltpu.SemaphoreType.DMA,
    ],
)
def _stage2(nodes_hbm, pw_hbm, out_hbm, nodes_v, buf0, buf1,
            gsem0, gsem1, ssem0, ssem1):
    _s2_body(nodes_hbm, pw_hbm, out_hbm, nodes_v, buf0, buf1,
             gsem0, gsem1, ssem0, ssem1)


def _prep_body(wi_ref, wf_ref, bi_ref, bf_ref, a_ref, c_ref, c0_ref):
    wi = wi_ref[...]
    wf1 = wf_ref[:D, :]
    wf2 = wf_ref[D:, :]
    a_ref[...] = jnp.dot(wi, wf1, preferred_element_type=jnp.float32)
    c_ref[...] = jnp.dot(wi, wf2, preferred_element_type=jnp.float32) * (
        1.0 / S)
    c0_ref[...] = (jnp.dot(bi_ref[...], wf1 + wf2,
                           preferred_element_type=jnp.float32) + bf_ref[...])


def _mm_body(feat_ref, nsum_ref, a_ref, c_ref, c0_ref, o_ref):
    x = jnp.dot(feat_ref[...], a_ref[...], preferred_element_type=jnp.float32)
    x = x + jnp.dot(nsum_ref[...], c_ref[...],
                    preferred_element_type=jnp.float32)
    x = x + c0_ref[...]
    o_ref[...] = x * (1.0 / (1.0 + jnp.exp(-x)))


def _fold_weights(w_init, w_final, b_init, b_final):
    return pl.pallas_call(
        _prep_body,
        out_shape=(jax.ShapeDtypeStruct((D, D), jnp.float32),
                   jax.ShapeDtypeStruct((D, D), jnp.float32),
                   jax.ShapeDtypeStruct((1, D), jnp.float32)),
    )(w_init, w_final, b_init.reshape(1, D), b_final.reshape(1, D))


def _matmul_swish(feats_p, nsum, a, c, c0):
    grid = (PAD // TB,)
    return pl.pallas_call(
        _mm_body,
        grid=grid,
        in_specs=[
            pl.BlockSpec((TB, D), lambda i: (i, 0)),
            pl.BlockSpec((TB, D), lambda i: (i, 0)),
            pl.BlockSpec((D, D), lambda i: (0, 0)),
            pl.BlockSpec((D, D), lambda i: (0, 0)),
            pl.BlockSpec((1, D), lambda i: (0, 0)),
        ],
        out_specs=pl.BlockSpec((TB, D), lambda i: (i, 0)),
        out_shape=jax.ShapeDtypeStruct((PAD, D), jnp.float32),
    )(feats_p, nsum, a, c, c0)


@jax.jit
def kernel(nodes, neigh_idx, features, W_init, b_init, W_final, b_final):
    nodes_p = jnp.pad(nodes.astype(jnp.int32), (0, PAD - BATCH))
    nidx_t = jnp.pad(neigh_idx.astype(jnp.int32),
                     ((0, PAD - N_NODES), (0, 0))).T
    nidx_w = nidx_t.reshape(S, NW, CPW * CH).transpose(1, 0, 2).reshape(
        NW, S * CPW, CH)
    nodes3 = nodes_p.reshape(NW, CPW, CH)
    feats_p = jnp.pad(features, ((0, PAD - N_NODES), (0, 0)))

    nsum = _stage1(nidx_w, features)
    a, c, c0 = _fold_weights(W_init, W_final, b_init, b_final)
    pw = _matmul_swish(feats_p, nsum, a, c, c0)
    out = _stage2(nodes3, pw)
    return out[:BATCH]


# speedup vs baseline: 1.0280x; 1.0087x over previous
import functools

import jax
import jax.numpy as jnp
from jax import lax
from jax.experimental import pallas as pl
from jax.experimental.pallas import tpu as pltpu
from jax.experimental.pallas import tpu_sc as plsc

N_NODES = 50000
D = 128
S = 10
BATCH = 50000

NC = 2
NS = 16
NW = NC * NS

CH = 128
CPW = 13
PER_W = CH * CPW
PAD = NW * PER_W

PLAN = ((0, 4, 0), (4, 2, 1), (6, 4, 0), (10, 2, 1), (12, 1, 1))

TB = 2000


def _worker_id():
    return lax.axis_index("s") * NC + lax.axis_index("c")


def _zero_rows(acc, nrows):
    zero = jnp.zeros((16,), jnp.float32)

    def body(r, _):
        for c in range(D // 16):
            acc[r, pl.ds(c * 16, 16)] = zero
        return 0

    lax.fori_loop(0, nrows, body, 0)


def _s1_body(nidx_hbm, feat_hbm, nsum_hbm, idx_v, acc0_v, acc1_v,
             gsem0, gsem1, ssem0, ssem1):
    wid = _worker_id()
    base_c = wid * CPW

    pltpu.sync_copy(nidx_hbm.at[wid], idx_v)

    accs = (acc0_v, acc1_v)
    gsems = (gsem0, gsem1)
    ssems = (ssem0, ssem1)
    store = [None, None]
    for g0, ck, b in PLAN:
        if store[b] is not None:
            store[b].wait()
        acc = accs[b]
        _zero_rows(acc, ck * CH)
        drains = []
        for c in range(ck):
            dst = acc.at[pl.ds(c * CH, CH)]
            for s in range(S):
                drains.append(pltpu.async_copy(
                    feat_hbm.at[idx_v.at[s * CPW + g0 + c]], dst, gsems[b],
                    add=True))
        for d in drains:
            d.wait()
        store[b] = pltpu.async_copy(
            acc.at[pl.ds(0, ck * CH)],
            nsum_hbm.at[pl.ds((base_c + g0) * CH, ck * CH)], ssems[b])
    for d in store:
        if d is not None:
            d.wait()


@functools.partial(
    pl.kernel,
    out_type=jax.ShapeDtypeStruct((PAD, D), jnp.float32),
    mesh=plsc.VectorSubcoreMesh(core_axis_name="c", subcore_axis_name="s"),
    scratch_types=[
        pltpu.VMEM((S * CPW, CH), jnp.int32),
        pltpu.VMEM((4 * CH, D), jnp.float32),
        pltpu.VMEM((2 * CH, D), jnp.float32),
        pltpu.SemaphoreType.DMA,
        pltpu.SemaphoreType.DMA,
        pltpu.SemaphoreType.DMA,
        pltpu.SemaphoreType.DMA,
    ],
)
def _stage1(nidx_hbm, feat_hbm, nsum_hbm, idx_v, acc0_v, acc1_v,
            gsem0, gsem1, ssem0, ssem1):
    _s1_body(nidx_hbm, feat_hbm, nsum_hbm, idx_v, acc0_v, acc1_v,
             gsem0, gsem1, ssem0, ssem1)


def _s2_body(nodes_hbm, pw_hbm, out_hbm, nodes_v, buf0, buf1,
             gsem0, gsem1, ssem0, ssem1):
    wid = _worker_id()
    base_c = wid * CPW

    pltpu.sync_copy(nodes_hbm.at[wid], nodes_v)

    bufs = (buf0, buf1)
    gsems = (gsem0, gsem1)
    ssems = (ssem0, ssem1)
    store = [None, None]
    for g0, ck, b in PLAN:
        if store[b] is not None:
            store[b].wait()
        buf = bufs[b]
        drains = []
        for c in range(ck):
            drains.append(pltpu.async_copy(
                pw_hbm.at[nodes_v.at[g0 + c]],
                buf.at[pl.ds(c * CH, CH)], gsems[b]))
        for d in drains:
            d.wait()
        store[b] = pltpu.async_copy(
            buf.at[pl.ds(0, ck * CH)],
            out_hbm.at[pl.ds((base_c + g0) * CH, ck * CH)], ssems[b])
    for d in store:
        if d is not None:
            d.wait()


@functools.partial(
    pl.kernel,
    out_type=jax.ShapeDtypeStruct((PAD, D), jnp.float32),
    mesh=plsc.VectorSubcoreMesh(core_axis_name="c", subcore_axis_name="s"),
    scratch_types=[
        pltpu.VMEM((CPW, CH), jnp.int32),
        pltpu.VMEM((4 * CH, D), jnp.float32),
        pltpu.VMEM((2 * CH, D), jnp.float32),
        pltpu.SemaphoreType.DMA,
        pltpu.SemaphoreType.DMA,
        pltpu.SemaphoreType.DMA,
        pltpu.SemaphoreType.DMA,
    ],
)
def _stage2(nodes_hbm, pw_hbm, out_hbm, nodes_v, buf0, buf1,
            gsem0, gsem1, ssem0, ssem1):
    _s2_body(nodes_hbm, pw_hbm, out_hbm, nodes_v, buf0, buf1,
             gsem0, gsem1, ssem0, ssem1)


def _prep_body(wi_ref, wf_ref, bi_ref, bf_ref, a_ref, c_ref, c0_ref):
    wi = wi_ref[...]
    wf1 = wf_ref[:D, :]
    wf2 = wf_ref[D:, :]
    a_ref[...] = jnp.dot(wi, wf1, preferred_element_type=jnp.float32)
    c_ref[...] = jnp.dot(wi, wf2, preferred_element_type=jnp.float32) * (
        1.0 / S)
    c0_ref[...] = (jnp.dot(bi_ref[...], wf1 + wf2,
                           preferred_element_type=jnp.float32) + bf_ref[...])


def _mm_body(feat_ref, nsum_ref, a_ref, c_ref, c0_ref, o_ref):
    x = jnp.dot(feat_ref[...], a_ref[...], preferred_element_type=jnp.float32)
    x = x + jnp.dot(nsum_ref[...], c_ref[...],
                    preferred_element_type=jnp.float32)
    x = x + c0_ref[...]
    o_ref[...] = x * (1.0 / (1.0 + jnp.exp(-x)))


def _fold_weights(w_init, w_final, b_init, b_final):
    return pl.pallas_call(
        _prep_body,
        out_shape=(jax.ShapeDtypeStruct((D, D), jnp.float32),
                   jax.ShapeDtypeStruct((D, D), jnp.float32),
                   jax.ShapeDtypeStruct((1, D), jnp.float32)),
    )(w_init, w_final, b_init.reshape(1, D), b_final.reshape(1, D))


def _matmul_swish(features, nsum, a, c, c0):
    grid = (N_NODES // TB,)
    return pl.pallas_call(
        _mm_body,
        grid=grid,
        in_specs=[
            pl.BlockSpec((TB, D), lambda i: (i, 0)),
            pl.BlockSpec((TB, D), lambda i: (i, 0)),
            pl.BlockSpec((D, D), lambda i: (0, 0)),
            pl.BlockSpec((D, D), lambda i: (0, 0)),
            pl.BlockSpec((1, D), lambda i: (0, 0)),
        ],
        out_specs=pl.BlockSpec((TB, D), lambda i: (i, 0)),
        out_shape=jax.ShapeDtypeStruct((N_NODES, D), jnp.float32),
    )(features, nsum, a, c, c0)


@jax.jit
def kernel(nodes, neigh_idx, features, W_init, b_init, W_final, b_final):
    nodes_p = jnp.pad(nodes.astype(jnp.int32), (0, PAD - BATCH))
    nidx_t = jnp.pad(neigh_idx.astype(jnp.int32),
                     ((0, PAD - N_NODES), (0, 0))).T
    nidx_w = nidx_t.reshape(S, NW, CPW * CH).transpose(1, 0, 2).reshape(
        NW, S * CPW, CH)
    nodes3 = nodes_p.reshape(NW, CPW, CH)

    nsum = _stage1(nidx_w, features)
    a, c, c0 = _fold_weights(W_init, W_final, b_init, b_final)
    pw = _matmul_swish(features, nsum, a, c, c0)
    out = _stage2(nodes3, pw)
    return out[:BATCH]
